# baseline (speedup 1.0000x reference)
import functools

import jax
import jax.numpy as jnp
from jax import lax
from jax.experimental import pallas as pl
from jax.experimental.pallas import tpu as pltpu
from jax.experimental.pallas import tpu_sc as plsc

ALPHA, BETA = 0.2, 0.8
NC, NS = 2, 16
CHUNK = 125
DEPTH = 8
LA = 4
DEGW = 16

_F32 = jnp.float32


def _mesh():
    return plsc.VectorSubcoreMesh(core_axis_name="c", subcore_axis_name="s",
                                  num_cores=NC, num_subcores=NS)


_SC_PARAMS = pltpu.CompilerParams(use_tc_tiling_on_sc=False)


def _row_split(n):
    rps = (n // 8 // NS) * 8
    return rps, rps * NS, n - rps * NS


def _zero_acc(zeros_hbm, acc, s, rps, tail_base, tail):
    pltpu.sync_copy(zeros_hbm.at[pl.ds(s * rps, rps)],
                    acc.at[pl.ds(s * rps, rps)])
    if tail:
        @pl.when(s == NS - 1)
        def _():
            pltpu.sync_copy(zeros_hbm.at[pl.ds(tail_base, tail)],
                            acc.at[pl.ds(tail_base, tail)])


def _copy_out(acc, out_c, s, rps, tail_base, tail):
    pltpu.sync_copy(acc.at[pl.ds(s * rps, rps)],
                    out_c.at[pl.ds(s * rps, rps)])
    if tail:
        @pl.when(s == NS - 1)
        def _():
            pltpu.sync_copy(acc.at[pl.ds(tail_base, tail)],
                            out_c.at[pl.ds(tail_base, tail)])


def _make_deg_kernel(n, e):
    rows_per_sub = e // CHUNK // NS
    ngroups = rows_per_sub // DEPTH
    rps, tail_base, tail = _row_split(n)

    @functools.partial(
        pl.kernel,
        out_type=jax.ShapeDtypeStruct((NC, n, DEGW), _F32),
        mesh=_mesh(),
        compiler_params=_SC_PARAMS,
        scratch_types=[
            pltpu.VMEM((rows_per_sub, CHUNK), jnp.int32),
            pltpu.VMEM((CHUNK, DEGW), _F32),
            pltpu.VMEM_SHARED((n, DEGW), _F32),
        ] + [pltpu.SemaphoreType.DMA] * DEPTH,
    )
    def deg_kernel(ei3d, zeros_hbm, ones_hbm, out, slab, ones_v, acc,
                   *ssems):
        c = lax.axis_index("c")
        s = lax.axis_index("s")
        _zero_acc(zeros_hbm, acc, s, rps, tail_base, tail)
        pltpu.sync_copy(ones_hbm, ones_v)
        row0 = s * rows_per_sub
        pltpu.sync_copy(ei3d.at[c, pl.ds(row0, rows_per_sub)], slab)

        plsc.subcore_barrier()

        def group(i, carry):
            for k in range(DEPTH):
                j = i * DEPTH + k

                @pl.when(j >= DEPTH)
                def _():
                    pltpu.make_async_copy(
                        ones_v, acc.at[pl.ds(0, CHUNK)], ssems[k]).wait()
                pltpu.async_copy(ones_v, acc.at[slab.at[j]], ssems[k],
                                 add=True)
            return carry
        lax.fori_loop(0, ngroups, group, 0)
        for k in range(DEPTH):
            pltpu.make_async_copy(
                ones_v, acc.at[pl.ds(0, CHUNK)], ssems[k]).wait()

        plsc.subcore_barrier()
        _copy_out(acc, out.at[c], s, rps, tail_base, tail)

    return deg_kernel


def _make_prop_half_kernel(n, e, f, sdim):
    rows_per_sub = e // CHUNK // NC // NS
    nch = rows_per_sub
    ngroups = nch // DEPTH
    rps, tail_base, tail = _row_split(n)

    @functools.partial(
        pl.kernel,
        out_type=jax.ShapeDtypeStruct((NC, n, f), _F32),
        mesh=_mesh(),
        compiler_params=_SC_PARAMS,
        scratch_types=[
            pltpu.VMEM((nch, CHUNK), jnp.int32),
            pltpu.VMEM((nch, CHUNK), jnp.int32),
        ] + [pltpu.VMEM((CHUNK, f), _F32)] * DEPTH + [
            pltpu.VMEM_SHARED((n, f), _F32),
        ] + [pltpu.SemaphoreType.DMA] * (2 * DEPTH),
    )
    def prop_half(tab, ei3d, zeros_hbm, out, idxs, idxd,
                  b0, b1, b2, b3, b4, b5, b6, b7, acc, *sems):
        bufs = (b0, b1, b2, b3, b4, b5, b6, b7)
        gsems = sems[:DEPTH]
        ssems = sems[DEPTH:]
        c = lax.axis_index("c")
        s = lax.axis_index("s")
        _zero_acc(zeros_hbm, acc, s, rps, tail_base, tail)
        row0 = (c * NS + s) * nch
        pltpu.sync_copy(ei3d.at[sdim, pl.ds(row0, nch)], idxs)
        pltpu.sync_copy(ei3d.at[1 - sdim, pl.ds(row0, nch)], idxd)
        plsc.subcore_barrier()

        for k in range(LA):
            pltpu.async_copy(tab.at[idxs.at[k]], bufs[k], gsems[k])

        def group(i, carry):
            for k in range(DEPTH):
                j = i * DEPTH + k
                k2 = (k + LA) % DEPTH
                pltpu.make_async_copy(
                    zeros_hbm.at[pl.ds(0, CHUNK)], bufs[k], gsems[k]).wait()
                pltpu.async_copy(bufs[k], acc.at[idxd.at[j]], ssems[k],
                                 add=True)
                @pl.when(j + LA < nch)
                def _():
                    @pl.when(j >= DEPTH - LA)
                    def _():
                        pltpu.make_async_copy(
                            bufs[k2], acc.at[pl.ds(0, CHUNK)],
                            ssems[k2]).wait()
                    pltpu.async_copy(tab.at[idxs.at[j + LA]], bufs[k2],
                                     gsems[k2])
            return carry
        lax.fori_loop(0, ngroups, group, 0)
        for k in range(DEPTH):
            pltpu.make_async_copy(
                bufs[k], acc.at[pl.ds(0, CHUNK)], ssems[k]).wait()

        plsc.subcore_barrier()
        _copy_out(acc, out.at[c], s, rps, tail_base, tail)

    return prop_half


def _ipow(deg, p):
    return jnp.exp(p * jnp.log(deg))


def _tc1a_body(x_ref, wi_ref, bi_ref, wt1_ref, bt1_ref, htr_ref):
    u = jnp.dot(x_ref[...], wi_ref[...], preferred_element_type=_F32) + bi_ref[...]
    htr_ref[...] = jnp.dot(u, wt1_ref[...], preferred_element_type=_F32) + bt1_ref[...]


def _tc1b_body(htr_ref, deg_ref, htt_ref):
    din = deg_ref[1][:, 0:1] + 1.0
    htt_ref[...] = _ipow(din, -BETA) * htr_ref[...]


def _tc2_body(q_ref, htt_ref, deg_ref, ws2_ref, bs2_ref, h2s_ref):
    dout = deg_ref[0][:, 0:1] + 1.0
    t1 = jnp.maximum(
        _ipow(dout, -ALPHA) * (q_ref[0] + q_ref[1] + htt_ref[...]), 0.0)
    h2s = jnp.dot(t1, ws2_ref[...], preferred_element_type=_F32) + bs2_ref[...]
    h2s_ref[...] = _ipow(dout, -BETA) * h2s


def _tc3_body(p_ref, h2s_ref, deg_ref, wt2_ref, bt2_ref,
              s2_ref, h2t_ref):
    din = deg_ref[1][:, 0:1] + 1.0
    s2 = _ipow(din, -ALPHA) * (p_ref[0] + p_ref[1] + h2s_ref[...])
    s2_ref[...] = s2
    h2t = jnp.dot(s2, wt2_ref[...], preferred_element_type=_F32) + bt2_ref[...]
    h2t_ref[...] = _ipow(din, -BETA) * h2t


def _tc4_body(q_ref, h2t_ref, deg_ref, t2_ref):
    dout = deg_ref[0][:, 0:1] + 1.0
    t2_ref[...] = _ipow(dout, -ALPHA) * (q_ref[0] + q_ref[1] + h2t_ref[...])


def kernel(x, edge_index, W_init, b_init, W_s1, b_s1, W_t1, b_t1,
           W_s2, b_s2, W_t2, b_t2):
    n, d = x.shape
    e = edge_index.shape[1]
    h = W_s1.shape[1]
    o = W_s2.shape[1]

    ei3d = edge_index.reshape(2, -1, CHUNK)
    zeros_deg = jnp.zeros((n, DEGW), _F32)
    ones_chunk = jnp.ones((CHUNK, DEGW), _F32)
    zeros_f = jnp.zeros((n, h), _F32)

    bi = b_init.reshape(1, d)
    bt1 = b_t1.reshape(1, h)
    bs2 = b_s2.reshape(1, o)
    bt2 = b_t2.reshape(1, o)

    deg = _make_deg_kernel(n, e)(ei3d, zeros_deg, ones_chunk)

    htr = pl.pallas_call(
        _tc1a_body,
        out_shape=jax.ShapeDtypeStruct((n, h), _F32),
    )(x, W_init, bi, W_t1, bt1)

    htt = pl.pallas_call(
        _tc1b_body,
        out_shape=jax.ShapeDtypeStruct((n, h), _F32),
    )(htr, deg)

    prop_rev = _make_prop_half_kernel(n, e, h, 1)
    prop_fwd = _make_prop_half_kernel(n, e, h, 0)

    parts_t1 = prop_rev(htt, ei3d, zeros_f)

    h2s = pl.pallas_call(
        _tc2_body,
        out_shape=jax.ShapeDtypeStruct((n, o), _F32),
    )(parts_t1, htt, deg, W_s2, bs2)

    parts_s = prop_fwd(h2s, ei3d, zeros_f)

    s2, h2t = pl.pallas_call(
        _tc3_body,
        out_shape=(jax.ShapeDtypeStruct((n, o), _F32),
                   jax.ShapeDtypeStruct((n, o), _F32)),
    )(parts_s, h2s, deg, W_t2, bt2)

    parts_t2 = prop_rev(h2t, ei3d, zeros_f)

    t2 = pl.pallas_call(
        _tc4_body,
        out_shape=jax.ShapeDtypeStruct((n, o), _F32),
    )(parts_t2, h2t, deg)

    return jnp.concatenate([s2, t2], axis=1)


# speedup vs baseline: 1.0318x; 1.0318x over previous
import functools

import jax
import jax.numpy as jnp
from jax import lax
from jax.experimental import pallas as pl
from jax.experimental.pallas import tpu as pltpu
from jax.experimental.pallas import tpu_sc as plsc

ALPHA, BETA = 0.2, 0.8
NC, NS = 2, 16
CHUNK = 125
DEPTH = 5
LA = 3
DEGW = 16

_F32 = jnp.float32


def _mesh():
    return plsc.VectorSubcoreMesh(core_axis_name="c", subcore_axis_name="s",
                                  num_cores=NC, num_subcores=NS)


_SC_PARAMS = pltpu.CompilerParams(use_tc_tiling_on_sc=False)


def _row_split(n):
    rps = (n // 8 // NS) * 8
    return rps, rps * NS, n - rps * NS


def _zero_acc(zeros_hbm, acc, s, rps, tail_base, tail):
    pltpu.sync_copy(zeros_hbm.at[pl.ds(s * rps, rps)],
                    acc.at[pl.ds(s * rps, rps)])
    if tail:
        @pl.when(s == NS - 1)
        def _():
            pltpu.sync_copy(zeros_hbm.at[pl.ds(tail_base, tail)],
                            acc.at[pl.ds(tail_base, tail)])


def _copy_out(acc, out_c, s, rps, tail_base, tail):
    pltpu.sync_copy(acc.at[pl.ds(s * rps, rps)],
                    out_c.at[pl.ds(s * rps, rps)])
    if tail:
        @pl.when(s == NS - 1)
        def _():
            pltpu.sync_copy(acc.at[pl.ds(tail_base, tail)],
                            out_c.at[pl.ds(tail_base, tail)])


def _make_deg_kernel(n, e):
    rows_per_sub = e // CHUNK // NS
    ngroups = rows_per_sub // DEPTH
    rps, tail_base, tail = _row_split(n)

    @functools.partial(
        pl.kernel,
        out_type=jax.ShapeDtypeStruct((NC, n, DEGW), _F32),
        mesh=_mesh(),
        compiler_params=_SC_PARAMS,
        scratch_types=[
            pltpu.VMEM((rows_per_sub, CHUNK), jnp.int32),
            pltpu.VMEM((CHUNK, DEGW), _F32),
            pltpu.VMEM_SHARED((n, DEGW), _F32),
        ] + [pltpu.SemaphoreType.DMA] * DEPTH,
    )
    def deg_kernel(ei3d, zeros_hbm, ones_hbm, out, slab, ones_v, acc,
                   *ssems):
        c = lax.axis_index("c")
        s = lax.axis_index("s")
        _zero_acc(zeros_hbm, acc, s, rps, tail_base, tail)
        pltpu.sync_copy(ones_hbm, ones_v)
        row0 = s * rows_per_sub
        pltpu.sync_copy(ei3d.at[c, pl.ds(row0, rows_per_sub)], slab)

        plsc.subcore_barrier()

        def group(i, carry):
            for k in range(DEPTH):
                j = i * DEPTH + k

                @pl.when(j >= DEPTH)
                def _():
                    pltpu.make_async_copy(
                        ones_v, acc.at[pl.ds(0, CHUNK)], ssems[k]).wait()
                pltpu.async_copy(ones_v, acc.at[slab.at[j]], ssems[k],
                                 add=True)
            return carry
        lax.fori_loop(0, ngroups, group, 0)
        for k in range(DEPTH):
            pltpu.make_async_copy(
                ones_v, acc.at[pl.ds(0, CHUNK)], ssems[k]).wait()

        plsc.subcore_barrier()
        _copy_out(acc, out.at[c], s, rps, tail_base, tail)

    return deg_kernel


def _make_prop_half_kernel(n, e, f, sdim):
    rows_per_sub = e // CHUNK // NC // NS
    nch = rows_per_sub
    ngroups = nch // DEPTH
    rps, tail_base, tail = _row_split(n)

    @functools.partial(
        pl.kernel,
        out_type=jax.ShapeDtypeStruct((NC, n, f), _F32),
        mesh=_mesh(),
        compiler_params=_SC_PARAMS,
        scratch_types=[
            pltpu.VMEM((nch, CHUNK), jnp.int32),
            pltpu.VMEM((nch, CHUNK), jnp.int32),
        ] + [pltpu.VMEM((CHUNK, f), _F32)] * DEPTH + [
            pltpu.VMEM_SHARED((n, f), _F32),
        ] + [pltpu.SemaphoreType.DMA] * (2 * DEPTH),
    )
    def prop_half(tab, ei3d, zeros_hbm, out, idxs, idxd,
                  b0, b1, b2, b3, b4, acc, *sems):
        bufs = (b0, b1, b2, b3, b4)
        gsems = sems[:DEPTH]
        ssems = sems[DEPTH:]
        c = lax.axis_index("c")
        s = lax.axis_index("s")
        _zero_acc(zeros_hbm, acc, s, rps, tail_base, tail)
        row0 = (c * NS + s) * nch
        pltpu.sync_copy(ei3d.at[sdim, pl.ds(row0, nch)], idxs)
        pltpu.sync_copy(ei3d.at[1 - sdim, pl.ds(row0, nch)], idxd)
        plsc.subcore_barrier()

        for k in range(LA):
            pltpu.async_copy(tab.at[idxs.at[k]], bufs[k], gsems[k])

        def group(i, carry):
            for k in range(DEPTH):
                j = i * DEPTH + k
                k2 = (k + LA) % DEPTH
                pltpu.make_async_copy(
                    zeros_hbm.at[pl.ds(0, CHUNK)], bufs[k], gsems[k]).wait()
                pltpu.async_copy(bufs[k], acc.at[idxd.at[j]], ssems[k],
                                 add=True)
                @pl.when(j + LA < nch)
                def _():
                    @pl.when(j >= DEPTH - LA)
                    def _():
                        pltpu.make_async_copy(
                            bufs[k2], acc.at[pl.ds(0, CHUNK)],
                            ssems[k2]).wait()
                    pltpu.async_copy(tab.at[idxs.at[j + LA]], bufs[k2],
                                     gsems[k2])
            return carry
        lax.fori_loop(0, ngroups, group, 0)
        for k in range(DEPTH):
            pltpu.make_async_copy(
                bufs[k], acc.at[pl.ds(0, CHUNK)], ssems[k]).wait()

        plsc.subcore_barrier()
        _copy_out(acc, out.at[c], s, rps, tail_base, tail)

    return prop_half


def _ipow(deg, p):
    return jnp.exp(p * jnp.log(deg))


def _tc1a_body(x_ref, wi_ref, bi_ref, wt1_ref, bt1_ref, htr_ref):
    u = jnp.dot(x_ref[...], wi_ref[...], preferred_element_type=_F32) + bi_ref[...]
    htr_ref[...] = jnp.dot(u, wt1_ref[...], preferred_element_type=_F32) + bt1_ref[...]


def _tc1b_body(htr_ref, deg_ref, htt_ref, sc_ref):
    din = deg_ref[1][:, 0:1] + 1.0
    dout = deg_ref[0][:, 0:1] + 1.0
    n = htr_ref.shape[0]
    da, db = _ipow(din, -ALPHA), _ipow(din, -BETA)
    oa, ob = _ipow(dout, -ALPHA), _ipow(dout, -BETA)
    sc_ref[...] = jnp.concatenate(
        [jnp.broadcast_to(v, (n, 32)) for v in (da, db, oa, ob)], axis=1)
    htt_ref[...] = db * htr_ref[...]


def _tc2_body(q_ref, htt_ref, sc_ref, ws2_ref, bs2_ref, h2s_ref):
    oa, ob = sc_ref[:, 64:65], sc_ref[:, 96:97]
    t1 = jnp.maximum(oa * (q_ref[0] + q_ref[1] + htt_ref[...]), 0.0)
    h2s = jnp.dot(t1, ws2_ref[...], preferred_element_type=_F32) + bs2_ref[...]
    h2s_ref[...] = ob * h2s


def _tc3_body(p_ref, h2s_ref, sc_ref, wt2_ref, bt2_ref,
              s2_ref, h2t_ref):
    da, db = sc_ref[:, 0:1], sc_ref[:, 32:33]
    s2 = da * (p_ref[0] + p_ref[1] + h2s_ref[...])
    s2_ref[...] = s2
    h2t = jnp.dot(s2, wt2_ref[...], preferred_element_type=_F32) + bt2_ref[...]
    h2t_ref[...] = db * h2t


def _tc4_body(q_ref, h2t_ref, sc_ref, s2_ref, out_ref):
    oa = sc_ref[:, 64:65]
    t2 = oa * (q_ref[0] + q_ref[1] + h2t_ref[...])
    out_ref[...] = jnp.concatenate([s2_ref[...], t2], axis=1)


def kernel(x, edge_index, W_init, b_init, W_s1, b_s1, W_t1, b_t1,
           W_s2, b_s2, W_t2, b_t2):
    n, d = x.shape
    e = edge_index.shape[1]
    h = W_s1.shape[1]
    o = W_s2.shape[1]

    ei3d = edge_index.reshape(2, -1, CHUNK)
    zeros_deg = jnp.zeros((n, DEGW), _F32)
    ones_chunk = jnp.ones((CHUNK, DEGW), _F32)
    zeros_f = jnp.zeros((n, h), _F32)

    bi = b_init.reshape(1, d)
    bt1 = b_t1.reshape(1, h)
    bs2 = b_s2.reshape(1, o)
    bt2 = b_t2.reshape(1, o)

    deg = _make_deg_kernel(n, e)(ei3d, zeros_deg, ones_chunk)

    htr = pl.pallas_call(
        _tc1a_body,
        out_shape=jax.ShapeDtypeStruct((n, h), _F32),
    )(x, W_init, bi, W_t1, bt1)

    htt, scales = pl.pallas_call(
        _tc1b_body,
        out_shape=(jax.ShapeDtypeStruct((n, h), _F32),
                   jax.ShapeDtypeStruct((n, 128), _F32)),
    )(htr, deg)

    prop_rev = _make_prop_half_kernel(n, e, h, 1)
    prop_fwd = _make_prop_half_kernel(n, e, h, 0)

    parts_t1 = prop_rev(htt, ei3d, zeros_f)

    h2s = pl.pallas_call(
        _tc2_body,
        out_shape=jax.ShapeDtypeStruct((n, o), _F32),
    )(parts_t1, htt, scales, W_s2, bs2)

    parts_s = prop_fwd(h2s, ei3d, zeros_f)

    s2, h2t = pl.pallas_call(
        _tc3_body,
        out_shape=(jax.ShapeDtypeStruct((n, o), _F32),
                   jax.ShapeDtypeStruct((n, o), _F32)),
    )(parts_s, h2s, scales, W_t2, bt2)

    parts_t2 = prop_rev(h2t, ei3d, zeros_f)

    return pl.pallas_call(
        _tc4_body,
        out_shape=jax.ShapeDtypeStruct((n, 2 * o), _F32),
    )(parts_t2, h2t, scales, s2)


# speedup vs baseline: 1.0322x; 1.0004x over previous
import functools

import jax
import jax.numpy as jnp
from jax import lax
from jax.experimental import pallas as pl
from jax.experimental.pallas import tpu as pltpu
from jax.experimental.pallas import tpu_sc as plsc

ALPHA, BETA = 0.2, 0.8
NC, NS = 2, 16
CHUNK = 125
DEPTH = 5
LA = 3
DEGW = 16

_F32 = jnp.float32


def _mesh():
    return plsc.VectorSubcoreMesh(core_axis_name="c", subcore_axis_name="s",
                                  num_cores=NC, num_subcores=NS)


_SC_PARAMS = pltpu.CompilerParams(use_tc_tiling_on_sc=False)


def _row_split(n):
    rps = (n // 8 // NS) * 8
    return rps, rps * NS, n - rps * NS


def _zero_acc(zeros_hbm, acc, s, rps, tail_base, tail):
    pltpu.sync_copy(zeros_hbm.at[pl.ds(s * rps, rps)],
                    acc.at[pl.ds(s * rps, rps)])
    if tail:
        @pl.when(s == NS - 1)
        def _():
            pltpu.sync_copy(zeros_hbm.at[pl.ds(tail_base, tail)],
                            acc.at[pl.ds(tail_base, tail)])


def _copy_out(acc, out_c, s, rps, tail_base, tail):
    pltpu.sync_copy(acc.at[pl.ds(s * rps, rps)],
                    out_c.at[pl.ds(s * rps, rps)])
    if tail:
        @pl.when(s == NS - 1)
        def _():
            pltpu.sync_copy(acc.at[pl.ds(tail_base, tail)],
                            out_c.at[pl.ds(tail_base, tail)])


def _make_deg_kernel(n, e):
    rows_per_sub = e // CHUNK // NS
    ngroups = rows_per_sub // DEPTH
    rps, tail_base, tail = _row_split(n)

    @functools.partial(
        pl.kernel,
        out_type=jax.ShapeDtypeStruct((NC, n, DEGW), _F32),
        mesh=_mesh(),
        compiler_params=_SC_PARAMS,
        scratch_types=[
            pltpu.VMEM((rows_per_sub, CHUNK), jnp.int32),
            pltpu.VMEM((CHUNK, DEGW), _F32),
            pltpu.VMEM_SHARED((n, DEGW), _F32),
        ] + [pltpu.SemaphoreType.DMA] * DEPTH,
    )
    def deg_kernel(ei3d, zeros_hbm, ones_hbm, out, slab, ones_v, acc,
                   *ssems):
        c = lax.axis_index("c")
        s = lax.axis_index("s")
        _zero_acc(zeros_hbm, acc, s, rps, tail_base, tail)
        pltpu.sync_copy(ones_hbm, ones_v)
        row0 = s * rows_per_sub
        pltpu.sync_copy(ei3d.at[c, pl.ds(row0, rows_per_sub)], slab)

        plsc.subcore_barrier()

        def group(i, carry):
            for k in range(DEPTH):
                j = i * DEPTH + k

                @pl.when(j >= DEPTH)
                def _():
                    pltpu.make_async_copy(
                        ones_v, acc.at[pl.ds(0, CHUNK)], ssems[k]).wait()
                pltpu.async_copy(ones_v, acc.at[slab.at[j]], ssems[k],
                                 add=True)
            return carry
        lax.fori_loop(0, ngroups, group, 0)
        for k in range(DEPTH):
            pltpu.make_async_copy(
                ones_v, acc.at[pl.ds(0, CHUNK)], ssems[k]).wait()

        plsc.subcore_barrier()
        _copy_out(acc, out.at[c], s, rps, tail_base, tail)

    return deg_kernel


def _make_prop_half_kernel(n, e, f, sdim):
    rows_per_sub = e // CHUNK // NC // NS
    nch = rows_per_sub
    assert nch % DEPTH == 0
    ngroups = nch // DEPTH
    rps, tail_base, tail = _row_split(n)

    @functools.partial(
        pl.kernel,
        out_type=jax.ShapeDtypeStruct((NC, n, f), _F32),
        mesh=_mesh(),
        compiler_params=_SC_PARAMS,
        scratch_types=[
            pltpu.VMEM((nch, CHUNK), jnp.int32),
            pltpu.VMEM((nch, CHUNK), jnp.int32),
        ] + [pltpu.VMEM((CHUNK, f), _F32)] * DEPTH + [
            pltpu.VMEM_SHARED((n, f), _F32),
        ] + [pltpu.SemaphoreType.DMA] * (2 * DEPTH),
    )
    def prop_half(tab, ei3d, zeros_hbm, out, idxs, idxd,
                  b0, b1, b2, b3, b4, acc, *sems):
        bufs = (b0, b1, b2, b3, b4)
        gsems = sems[:DEPTH]
        ssems = sems[DEPTH:]
        c = lax.axis_index("c")
        s = lax.axis_index("s")
        _zero_acc(zeros_hbm, acc, s, rps, tail_base, tail)
        row0 = (c * NS + s) * nch
        pltpu.sync_copy(ei3d.at[sdim, pl.ds(row0, nch)], idxs)
        pltpu.sync_copy(ei3d.at[1 - sdim, pl.ds(row0, nch)], idxd)
        plsc.subcore_barrier()

        for k in range(LA):
            pltpu.async_copy(tab.at[idxs.at[k]], bufs[k], gsems[k])

        def group(i, carry):
            for k in range(DEPTH):
                j = i * DEPTH + k
                k2 = (k + LA) % DEPTH
                pltpu.make_async_copy(
                    zeros_hbm.at[pl.ds(0, CHUNK)], bufs[k], gsems[k]).wait()
                pltpu.async_copy(bufs[k], acc.at[idxd.at[j]], ssems[k],
                                 add=True)
                @pl.when(j + LA < nch)
                def _():
                    @pl.when(j >= DEPTH - LA)
                    def _():
                        pltpu.make_async_copy(
                            bufs[k2], acc.at[pl.ds(0, CHUNK)],
                            ssems[k2]).wait()
                    pltpu.async_copy(tab.at[idxs.at[j + LA]], bufs[k2],
                                     gsems[k2])
            return carry
        lax.fori_loop(0, ngroups, group, 0)
        for k in range(DEPTH):
            pltpu.make_async_copy(
                bufs[k], acc.at[pl.ds(0, CHUNK)], ssems[k]).wait()

        plsc.subcore_barrier()
        _copy_out(acc, out.at[c], s, rps, tail_base, tail)

    return prop_half


def _ipow(deg, p):
    return jnp.exp(p * jnp.log(deg))


def _tc1a_body(x_ref, wi_ref, bi_ref, wt1_ref, bt1_ref, htr_ref):
    u = jnp.dot(x_ref[...], wi_ref[...], preferred_element_type=_F32) + bi_ref[...]
    htr_ref[...] = jnp.dot(u, wt1_ref[...], preferred_element_type=_F32) + bt1_ref[...]


def _tc1b_body(htr_ref, deg_ref, htt_ref, sc_ref):
    din = deg_ref[1][:, 0:1] + 1.0
    dout = deg_ref[0][:, 0:1] + 1.0
    n = htr_ref.shape[0]
    da, db = _ipow(din, -ALPHA), _ipow(din, -BETA)
    oa, ob = _ipow(dout, -ALPHA), _ipow(dout, -BETA)
    sc_ref[...] = jnp.concatenate(
        [jnp.broadcast_to(v, (n, 32)) for v in (da, db, oa, ob)], axis=1)
    htt_ref[...] = db * htr_ref[...]


def _tc2_body(q_ref, htt_ref, sc_ref, ws2_ref, bs2_ref, h2s_ref):
    oa, ob = sc_ref[:, 64:65], sc_ref[:, 96:97]
    t1 = jnp.maximum(oa * (q_ref[0] + q_ref[1] + htt_ref[...]), 0.0)
    h2s = jnp.dot(t1, ws2_ref[...], preferred_element_type=_F32) + bs2_ref[...]
    h2s_ref[...] = ob * h2s


def _tc3_body(p_ref, h2s_ref, sc_ref, wt2_ref, bt2_ref,
              s2_ref, h2t_ref):
    da, db = sc_ref[:, 0:1], sc_ref[:, 32:33]
    s2 = da * (p_ref[0] + p_ref[1] + h2s_ref[...])
    s2_ref[...] = s2
    h2t = jnp.dot(s2, wt2_ref[...], preferred_element_type=_F32) + bt2_ref[...]
    h2t_ref[...] = db * h2t


def _tc4_body(q_ref, h2t_ref, sc_ref, s2_ref, out_ref):
    oa = sc_ref[:, 64:65]
    t2 = oa * (q_ref[0] + q_ref[1] + h2t_ref[...])
    out_ref[...] = jnp.concatenate([s2_ref[...], t2], axis=1)


def kernel(x, edge_index, W_init, b_init, W_s1, b_s1, W_t1, b_t1,
           W_s2, b_s2, W_t2, b_t2):
    n, d = x.shape
    e = edge_index.shape[1]
    h = W_s1.shape[1]
    o = W_s2.shape[1]

    ei3d = edge_index.reshape(2, -1, CHUNK)
    zeros_deg = jnp.zeros((n, DEGW), _F32)
    ones_chunk = jnp.ones((CHUNK, DEGW), _F32)
    zeros_f = jnp.zeros((n, h), _F32)

    bi = b_init.reshape(1, d)
    bt1 = b_t1.reshape(1, h)
    bs2 = b_s2.reshape(1, o)
    bt2 = b_t2.reshape(1, o)

    deg = _make_deg_kernel(n, e)(ei3d, zeros_deg, ones_chunk)

    htr = pl.pallas_call(
        _tc1a_body,
        out_shape=jax.ShapeDtypeStruct((n, h), _F32),
    )(x, W_init, bi, W_t1, bt1)

    htt, scales = pl.pallas_call(
        _tc1b_body,
        out_shape=(jax.ShapeDtypeStruct((n, h), _F32),
                   jax.ShapeDtypeStruct((n, 128), _F32)),
    )(htr, deg)

    prop_rev = _make_prop_half_kernel(n, e, h, 1)
    prop_fwd = _make_prop_half_kernel(n, e, h, 0)

    parts_t1 = prop_rev(htt, ei3d, zeros_f)

    h2s = pl.pallas_call(
        _tc2_body,
        out_shape=jax.ShapeDtypeStruct((n, o), _F32),
    )(parts_t1, htt, scales, W_s2, bs2)

    parts_s = prop_fwd(h2s, ei3d, zeros_f)

    s2, h2t = pl.pallas_call(
        _tc3_body,
        out_shape=(jax.ShapeDtypeStruct((n, o), _F32),
                   jax.ShapeDtypeStruct((n, o), _F32)),
    )(parts_s, h2s, scales, W_t2, bt2)

    parts_t2 = prop_rev(h2t, ei3d, zeros_f)

    return pl.pallas_call(
        _tc4_body,
        out_shape=jax.ShapeDtypeStruct((n, 2 * o), _F32),
    )(parts_t2, h2t, scales, s2)


# speedup vs baseline: 1.0520x; 1.0192x over previous
import functools

import jax
import jax.numpy as jnp
from jax import lax
from jax.experimental import pallas as pl
from jax.experimental.pallas import tpu as pltpu
from jax.experimental.pallas import tpu_sc as plsc

ALPHA, BETA = 0.2, 0.8
NC, NS = 2, 16
CHUNK = 125
DEPTH = 5
LA = 3
DEGW = 8

_F32 = jnp.float32


def _mesh():
    return plsc.VectorSubcoreMesh(core_axis_name="c", subcore_axis_name="s",
                                  num_cores=NC, num_subcores=NS)


_SC_PARAMS = pltpu.CompilerParams(use_tc_tiling_on_sc=False)


def _row_split(n):
    rps = (n // 8 // NS) * 8
    return rps, rps * NS, n - rps * NS


def _zero_acc(zeros_hbm, acc, s, rps, tail_base, tail):
    pltpu.sync_copy(zeros_hbm.at[pl.ds(s * rps, rps)],
                    acc.at[pl.ds(s * rps, rps)])
    if tail:
        @pl.when(s == NS - 1)
        def _():
            pltpu.sync_copy(zeros_hbm.at[pl.ds(tail_base, tail)],
                            acc.at[pl.ds(tail_base, tail)])


def _copy_out(acc, out_c, s, rps, tail_base, tail):
    pltpu.sync_copy(acc.at[pl.ds(s * rps, rps)],
                    out_c.at[pl.ds(s * rps, rps)])
    if tail:
        @pl.when(s == NS - 1)
        def _():
            pltpu.sync_copy(acc.at[pl.ds(tail_base, tail)],
                            out_c.at[pl.ds(tail_base, tail)])


def _make_deg_kernel(n, e):
    rows_per_sub = e // CHUNK // NS
    ngroups = rows_per_sub // DEPTH
    rps, tail_base, tail = _row_split(n)

    @functools.partial(
        pl.kernel,
        out_type=jax.ShapeDtypeStruct((NC, n, DEGW), _F32),
        mesh=_mesh(),
        compiler_params=_SC_PARAMS,
        scratch_types=[
            pltpu.VMEM((rows_per_sub, CHUNK), jnp.int32),
            pltpu.VMEM((CHUNK, DEGW), _F32),
            pltpu.VMEM_SHARED((n, DEGW), _F32),
        ] + [pltpu.SemaphoreType.DMA] * DEPTH,
    )
    def deg_kernel(ei3d, zeros_hbm, ones_hbm, out, slab, ones_v, acc,
                   *ssems):
        c = lax.axis_index("c")
        s = lax.axis_index("s")
        _zero_acc(zeros_hbm, acc, s, rps, tail_base, tail)
        pltpu.sync_copy(ones_hbm, ones_v)
        row0 = s * rows_per_sub
        pltpu.sync_copy(ei3d.at[c, pl.ds(row0, rows_per_sub)], slab)

        plsc.subcore_barrier()

        def group(i, carry):
            for k in range(DEPTH):
                j = i * DEPTH + k

                @pl.when(j >= DEPTH)
                def _():
                    pltpu.make_async_copy(
                        ones_v, acc.at[pl.ds(0, CHUNK)], ssems[k]).wait()
                pltpu.async_copy(ones_v, acc.at[slab.at[j]], ssems[k],
                                 add=True)
            return carry
        lax.fori_loop(0, ngroups, group, 0)
        for k in range(DEPTH):
            pltpu.make_async_copy(
                ones_v, acc.at[pl.ds(0, CHUNK)], ssems[k]).wait()

        plsc.subcore_barrier()
        _copy_out(acc, out.at[c], s, rps, tail_base, tail)

    return deg_kernel


def _make_prop_half_kernel(n, e, f, sdim):
    rows_per_sub = e // CHUNK // NC // NS
    nch = rows_per_sub
    assert nch % DEPTH == 0
    ngroups = nch // DEPTH
    rps, tail_base, tail = _row_split(n)

    @functools.partial(
        pl.kernel,
        out_type=jax.ShapeDtypeStruct((NC, n, f), _F32),
        mesh=_mesh(),
        compiler_params=_SC_PARAMS,
        scratch_types=[
            pltpu.VMEM((nch, CHUNK), jnp.int32),
            pltpu.VMEM((nch, CHUNK), jnp.int32),
        ] + [pltpu.VMEM((CHUNK, f), _F32)] * DEPTH + [
            pltpu.VMEM_SHARED((n, f), _F32),
        ] + [pltpu.SemaphoreType.DMA] * (2 * DEPTH),
    )
    def prop_half(tab, ei3d, zeros_hbm, out, idxs, idxd,
                  b0, b1, b2, b3, b4, acc, *sems):
        bufs = (b0, b1, b2, b3, b4)
        gsems = sems[:DEPTH]
        ssems = sems[DEPTH:]
        c = lax.axis_index("c")
        s = lax.axis_index("s")
        _zero_acc(zeros_hbm, acc, s, rps, tail_base, tail)
        row0 = (c * NS + s) * nch
        pltpu.sync_copy(ei3d.at[sdim, pl.ds(row0, nch)], idxs)
        pltpu.sync_copy(ei3d.at[1 - sdim, pl.ds(row0, nch)], idxd)
        plsc.subcore_barrier()

        for k in range(LA):
            pltpu.async_copy(tab.at[idxs.at[k]], bufs[k], gsems[k])

        def group(i, carry):
            for k in range(DEPTH):
                j = i * DEPTH + k
                k2 = (k + LA) % DEPTH
                pltpu.make_async_copy(
                    zeros_hbm.at[pl.ds(0, CHUNK)], bufs[k], gsems[k]).wait()
                pltpu.async_copy(bufs[k], acc.at[idxd.at[j]], ssems[k],
                                 add=True)
                @pl.when(j + LA < nch)
                def _():
                    @pl.when(j >= DEPTH - LA)
                    def _():
                        pltpu.make_async_copy(
                            bufs[k2], acc.at[pl.ds(0, CHUNK)],
                            ssems[k2]).wait()
                    pltpu.async_copy(tab.at[idxs.at[j + LA]], bufs[k2],
                                     gsems[k2])
            return carry
        lax.fori_loop(0, ngroups, group, 0)
        for k in range(DEPTH):
            pltpu.make_async_copy(
                bufs[k], acc.at[pl.ds(0, CHUNK)], ssems[k]).wait()

        plsc.subcore_barrier()
        _copy_out(acc, out.at[c], s, rps, tail_base, tail)

    return prop_half


def _ipow(deg, p):
    return jnp.exp(p * jnp.log(deg))


def _tc1a_body(x_ref, wi_ref, bi_ref, wt1_ref, bt1_ref, htr_ref):
    u = jnp.dot(x_ref[...], wi_ref[...], preferred_element_type=_F32) + bi_ref[...]
    htr_ref[...] = jnp.dot(u, wt1_ref[...], preferred_element_type=_F32) + bt1_ref[...]


def _tc1b_body(htr_ref, deg_ref, htt_ref, sc_ref):
    din = deg_ref[1][:, 0:1] + 1.0
    dout = deg_ref[0][:, 0:1] + 1.0
    n = htr_ref.shape[0]
    da, db = _ipow(din, -ALPHA), _ipow(din, -BETA)
    oa, ob = _ipow(dout, -ALPHA), _ipow(dout, -BETA)
    sc_ref[...] = jnp.concatenate(
        [jnp.broadcast_to(v, (n, 32)) for v in (da, db, oa, ob)], axis=1)
    htt_ref[...] = db * htr_ref[...]


def _tc2_body(q_ref, htt_ref, sc_ref, ws2_ref, bs2_ref, h2s_ref):
    oa, ob = sc_ref[:, 64:65], sc_ref[:, 96:97]
    t1 = jnp.maximum(oa * (q_ref[0] + q_ref[1] + htt_ref[...]), 0.0)
    h2s = jnp.dot(t1, ws2_ref[...], preferred_element_type=_F32) + bs2_ref[...]
    h2s_ref[...] = ob * h2s


def _tc3_body(p_ref, h2s_ref, sc_ref, wt2_ref, bt2_ref,
              s2_ref, h2t_ref):
    da, db = sc_ref[:, 0:1], sc_ref[:, 32:33]
    s2 = da * (p_ref[0] + p_ref[1] + h2s_ref[...])
    s2_ref[...] = s2
    h2t = jnp.dot(s2, wt2_ref[...], preferred_element_type=_F32) + bt2_ref[...]
    h2t_ref[...] = db * h2t


def _tc4_body(q_ref, h2t_ref, sc_ref, s2_ref, out_ref):
    oa = sc_ref[:, 64:65]
    t2 = oa * (q_ref[0] + q_ref[1] + h2t_ref[...])
    out_ref[...] = jnp.concatenate([s2_ref[...], t2], axis=1)


def kernel(x, edge_index, W_init, b_init, W_s1, b_s1, W_t1, b_t1,
           W_s2, b_s2, W_t2, b_t2):
    n, d = x.shape
    e = edge_index.shape[1]
    h = W_s1.shape[1]
    o = W_s2.shape[1]

    ei3d = edge_index.reshape(2, -1, CHUNK)
    zeros_deg = jnp.zeros((n, DEGW), _F32)
    ones_chunk = jnp.ones((CHUNK, DEGW), _F32)
    zeros_f = jnp.zeros((n, h), _F32)

    bi = b_init.reshape(1, d)
    bt1 = b_t1.reshape(1, h)
    bs2 = b_s2.reshape(1, o)
    bt2 = b_t2.reshape(1, o)

    deg = _make_deg_kernel(n, e)(ei3d, zeros_deg, ones_chunk)

    htr = pl.pallas_call(
        _tc1a_body,
        out_shape=jax.ShapeDtypeStruct((n, h), _F32),
    )(x, W_init, bi, W_t1, bt1)

    htt, scales = pl.pallas_call(
        _tc1b_body,
        out_shape=(jax.ShapeDtypeStruct((n, h), _F32),
                   jax.ShapeDtypeStruct((n, 128), _F32)),
    )(htr, deg)

    prop_rev = _make_prop_half_kernel(n, e, h, 1)
    prop_fwd = _make_prop_half_kernel(n, e, h, 0)

    parts_t1 = prop_rev(htt, ei3d, zeros_f)

    h2s = pl.pallas_call(
        _tc2_body,
        out_shape=jax.ShapeDtypeStruct((n, o), _F32),
    )(parts_t1, htt, scales, W_s2, bs2)

    parts_s = prop_fwd(h2s, ei3d, zeros_f)

    s2, h2t = pl.pallas_call(
        _tc3_body,
        out_shape=(jax.ShapeDtypeStruct((n, o), _F32),
                   jax.ShapeDtypeStruct((n, o), _F32)),
    )(parts_s, h2s, scales, W_t2, bt2)

    parts_t2 = prop_rev(h2t, ei3d, zeros_f)

    return pl.pallas_call(
        _tc4_body,
        out_shape=jax.ShapeDtypeStruct((n, 2 * o), _F32),
    )(parts_t2, h2t, scales, s2)
